# baseline (speedup 1.0000x reference)
import jax
import jax.numpy as jnp
from jax.experimental import pallas as pl
from jax.experimental.pallas import tpu as pltpu
from jax.experimental.pallas import tpu_sc as plsc

VOCAB = 32000
D = 1024
H = 16
DH = 64
DP = 128
HP = H * DP
FF = 4 * D
S = 2048

BF = jnp.bfloat16
F32 = jnp.float32


def _ln(x, g, b):
    mu = jnp.mean(x, axis=-1, keepdims=True)
    var = jnp.mean((x - mu) ** 2, axis=-1, keepdims=True)
    return (x - mu) * jax.lax.rsqrt(var + 1e-5) * g + b


_NC = 2
_NS = 16
_NW = _NC * _NS
_BPW = S // _NW


def _sc_gather(emb, ids):
    mesh = plsc.VectorSubcoreMesh(core_axis_name="c", subcore_axis_name="s")

    @pl.kernel(out_type=jax.ShapeDtypeStruct((S, D), emb.dtype), mesh=mesh,
               scratch_types=[
                   pltpu.VMEM((_BPW,), jnp.int32),
                   pltpu.VMEM((_BPW, D), jnp.float32),
                   pltpu.SemaphoreType.DMA,
               ])
    def k(emb_hbm, ids_hbm, o_hbm, idx_v, rows_v, sem):
        wid = jax.lax.axis_index("s") * _NC + jax.lax.axis_index("c")
        base = wid * _BPW
        pltpu.sync_copy(ids_hbm.at[pl.ds(base, _BPW)], idx_v)
        pltpu.async_copy(emb_hbm.at[idx_v], rows_v, sem).wait()
        pltpu.sync_copy(rows_v, o_hbm.at[pl.ds(base, _BPW)])

    return k(emb, ids.reshape(S))


def _lqkv_body(x_ref, wk_ref, wq_ref, wv_ref, g_ref, b_ref, bk_ref,
               bq_ref, bv_ref, kt_ref, q_ref, v_ref, wkt_ref):
    @pl.when(pl.program_id(0) == 0)
    def _():
        wkt_ref[...] = wk_ref[...].astype(BF).T

    h = _ln(x_ref[...], g_ref[...], b_ref[...]).astype(BF)
    ht = h.T
    kt = jnp.dot(wkt_ref[...], ht, preferred_element_type=F32)
    kt_ref[...] = (kt + bk_ref[...]).astype(BF)
    q_ref[...] = (jnp.dot(h, wq_ref[...].astype(BF), preferred_element_type=F32)
                  + bq_ref[...]).astype(BF)
    v_ref[...] = (jnp.dot(h, wv_ref[...].astype(BF), preferred_element_type=F32)
                  + bv_ref[...]).astype(BF)


def _lqkv(x, Wk, Wq, Wv, g, b, bk, bq, bv):
    SB = S // 2
    return pl.pallas_call(
        _lqkv_body,
        grid=(2,),
        in_specs=[
            pl.BlockSpec((SB, D), lambda i: (i, 0)),
            pl.BlockSpec((D, D), lambda i: (0, 0)),
            pl.BlockSpec((D, D), lambda i: (0, 0)),
            pl.BlockSpec((D, D), lambda i: (0, 0)),
            pl.BlockSpec((1, D), lambda i: (0, 0)),
            pl.BlockSpec((1, D), lambda i: (0, 0)),
            pl.BlockSpec((D, 1), lambda i: (0, 0)),
            pl.BlockSpec((1, D), lambda i: (0, 0)),
            pl.BlockSpec((1, D), lambda i: (0, 0)),
        ],
        out_specs=[pl.BlockSpec((D, SB), lambda i: (0, i)),
                   pl.BlockSpec((SB, D), lambda i: (i, 0)),
                   pl.BlockSpec((SB, D), lambda i: (i, 0))],
        out_shape=[jax.ShapeDtypeStruct((D, S), BF),
                   jax.ShapeDtypeStruct((S, D), BF),
                   jax.ShapeDtypeStruct((S, D), BF)],
        scratch_shapes=[pltpu.VMEM((D, D), BF)],
    )(x, Wk, Wq, Wv, g.reshape(1, D), b.reshape(1, D), bk.reshape(D, 1),
      bq.reshape(1, D), bv.reshape(1, D))


def _cast2_body(w1_ref, w2_ref, o1_ref, o2_ref):
    o1_ref[...] = w1_ref[...].astype(BF)
    o2_ref[...] = w2_ref[...].astype(BF)


def _cast_ffn_weights(W1, W2):
    FB = 1024
    return pl.pallas_call(
        _cast2_body,
        grid=(FF // FB,),
        in_specs=[
            pl.BlockSpec((D, FB), lambda j: (0, j)),
            pl.BlockSpec((FB, D), lambda j: (j, 0)),
        ],
        out_specs=[pl.BlockSpec((D, FB), lambda j: (0, j)),
                   pl.BlockSpec((FB, D), lambda j: (j, 0))],
        out_shape=[jax.ShapeDtypeStruct((D, FF), BF),
                   jax.ShapeDtypeStruct((FF, D), BF)],
    )(W1, W2)


_BQ = 2048


def _attn_body(q_ref, kt_ref, v_ref, o_ref):
    q = q_ref[...]
    kt = kt_ref[...]
    v = v_ref[...]
    outs = []
    for t in range(2):
        qh = q[:, t * DH:(t + 1) * DH]
        kth = kt[t * DH:(t + 1) * DH, :]
        s = jnp.dot(qh, kth, preferred_element_type=F32) * (1.0 / 8.0)
        p = jnp.exp(s.astype(BF))
        l = jnp.sum(p, axis=-1, keepdims=True, dtype=F32)
        vh = v[:, t * DH:(t + 1) * DH]
        o = jnp.dot(p, vh, preferred_element_type=F32)
        outs.append((o * (1.0 / l)).astype(BF))
    o_ref[...] = jnp.concatenate(outs, axis=1)


def _attn(q, kt, v):
    return pl.pallas_call(
        _attn_body,
        grid=(H // 2, S // _BQ),
        in_specs=[
            pl.BlockSpec((_BQ, DP), lambda h, i: (i, h)),
            pl.BlockSpec((DP, S), lambda h, i: (h, 0)),
            pl.BlockSpec((S, DP), lambda h, i: (0, h)),
        ],
        out_specs=pl.BlockSpec((_BQ, DP), lambda h, i: (i, h)),
        out_shape=jax.ShapeDtypeStruct((S, D), BF),
    )(q, kt, v)


def _proj_body(a_ref, wo_ref, bo_ref, x_ref, g_ref, b_ref, y_ref, h2_ref,
               wo_bf_ref):
    @pl.when(pl.program_id(0) == 0)
    def _():
        wo_bf_ref[...] = wo_ref[...].astype(BF)

    y = (x_ref[...]
         + jnp.dot(a_ref[...], wo_bf_ref[...], preferred_element_type=F32)
         + bo_ref[...])
    y_ref[...] = y
    h2_ref[...] = _ln(y, g_ref[...], b_ref[...]).astype(BF)


def _proj_ln2(a, Wo, bo, x, g, b):
    SB = S // 2
    return pl.pallas_call(
        _proj_body,
        grid=(2,),
        in_specs=[
            pl.BlockSpec((SB, D), lambda i: (i, 0)),
            pl.BlockSpec((D, D), lambda i: (0, 0)),
            pl.BlockSpec((1, D), lambda i: (0, 0)),
            pl.BlockSpec((SB, D), lambda i: (i, 0)),
            pl.BlockSpec((1, D), lambda i: (0, 0)),
            pl.BlockSpec((1, D), lambda i: (0, 0)),
        ],
        out_specs=[pl.BlockSpec((SB, D), lambda i: (i, 0))] * 2,
        out_shape=[jax.ShapeDtypeStruct((S, D), F32),
                   jax.ShapeDtypeStruct((S, D), BF)],
        scratch_shapes=[pltpu.VMEM((D, D), BF)],
    )(a, Wo, bo.reshape(1, D), x, g.reshape(1, D), b.reshape(1, D))


_FB = 1024


def _ffn_body(h2_ref, w1_ref, b1_ref, w2_ref, y_ref, b2_ref, o_ref, acc_ref):
    j = pl.program_id(0)
    t = jnp.dot(h2_ref[...], w1_ref[...], preferred_element_type=F32) + b1_ref[...]
    tb = jax.nn.gelu(t).astype(BF)
    part = jnp.dot(tb, w2_ref[...], preferred_element_type=F32)

    @pl.when(j == 0)
    def _():
        acc_ref[...] = part

    @pl.when(j > 0)
    def _():
        acc_ref[...] += part

    @pl.when(j == FF // _FB - 1)
    def _():
        o_ref[...] = (y_ref[...] + acc_ref[...] + b2_ref[...]).astype(BF)


def _ffn(h2, W1b, b1, W2b, y, b2):
    return pl.pallas_call(
        _ffn_body,
        grid=(FF // _FB,),
        in_specs=[
            pl.BlockSpec((S, D), lambda j: (0, 0)),
            pl.BlockSpec((D, _FB), lambda j: (0, j)),
            pl.BlockSpec((1, _FB), lambda j: (0, j)),
            pl.BlockSpec((_FB, D), lambda j: (j, 0)),
            pl.BlockSpec((S, D), lambda j: (0, 0)),
            pl.BlockSpec((1, D), lambda j: (0, 0)),
        ],
        out_specs=pl.BlockSpec((S, D), lambda j: (0, 0)),
        out_shape=jax.ShapeDtypeStruct((S, D), BF),
        scratch_shapes=[pltpu.VMEM((S, D), F32)],
    )(h2, W1b, b1.reshape(1, FF), W2b, y, b2.reshape(1, D))


def _dec_body(f_ref, w_ref, b_ref, o_ref):
    w = w_ref[...].astype(BF)
    o_ref[...] = jnp.dot(f_ref[...], w, preferred_element_type=F32) + b_ref[...]


def _decode(f, dec_W, dec_b):
    VB = 640
    return pl.pallas_call(
        _dec_body,
        grid=(VOCAB // VB,),
        in_specs=[
            pl.BlockSpec((S, D), lambda j: (0, 0)),
            pl.BlockSpec((D, VB), lambda j: (0, j)),
            pl.BlockSpec((1, VB), lambda j: (0, j)),
        ],
        out_specs=pl.BlockSpec((S, VB), lambda j: (0, j)),
        out_shape=jax.ShapeDtypeStruct((S, VOCAB), F32),
    )(f, dec_W, dec_b.reshape(1, VOCAB))


def _tc_forward(x, W1b, W2b, Wq, bq, Wk, bk, Wv, bv, Wo, bo, ln1_g, ln1_b,
                ln2_g, ln2_b, W1, b1, W2, b2, dec_W, dec_b):
    kt, q, v = _lqkv(x, Wk, Wq, Wv, ln1_g, ln1_b, bk, bq, bv)
    a = _attn(q, kt, v)
    y, h2 = _proj_ln2(a, Wo, bo, x, ln2_g, ln2_b)
    f = _ffn(h2, W1b, b1, W2b, y, b2)
    return _decode(f, dec_W, dec_b)


def kernel(input_ids, top_k, emb, ln1_g, ln1_b, Wq, bq, Wk, bk, Wv, bv,
           Wo, bo, ln2_g, ln2_b, W1, b1, W2, b2, dec_W, dec_b):
    ids = input_ids.reshape(1, S).astype(jnp.int32)
    W1b, W2b = _cast_ffn_weights(W1, W2)
    x = _sc_gather(emb, ids)
    logits = _tc_forward(x, W1b, W2b, Wq, bq, Wk, bk, Wv, bv, Wo, bo,
                         ln1_g, ln1_b, ln2_g, ln2_b, W1, b1, W2, b2,
                         dec_W, dec_b)
    return logits.reshape(1, S, VOCAB)
